# baseline (speedup 1.0000x reference)
import functools

import jax
import jax.numpy as jnp
from jax import lax
from jax.experimental import pallas as pl
from jax.experimental.pallas import tpu as pltpu
from jax.experimental.pallas import tpu_sc as plsc

S = 2048
D_MODEL = 768
D_FF = 2048
E = 8
CAP = 512
EPS = 1e-6
CHUNK = 256
FFB = 1024
NSLOT = E * CAP
TBL_ROWS = NSLOT + S

_LANES = 16
_NW = 32
_TPW = S // _NW


def _router_body(x_ref, lnw_ref, cls_ref, fwd_ref, prep_ref, gidx_ref):
    x = x_ref[...]
    var = jnp.mean(x * x, axis=1, keepdims=True)
    fwd = x * lax.rsqrt(var + EPS) * lnw_ref[...]
    fwd_ref[...] = fwd

    logits = lax.dot_general(fwd, cls_ref[...], (((1,), (1,)), ((), ())),
                             preferred_element_type=jnp.float32)
    m = jnp.max(logits, axis=1, keepdims=True)
    ex = jnp.exp(logits - m)
    probs = ex / jnp.sum(ex, axis=1, keepdims=True)
    pmax = jnp.max(probs, axis=1, keepdims=True)
    prep_ref[...] = jnp.broadcast_to(pmax, (S, _LANES))

    lane = lax.broadcasted_iota(jnp.int32, (S, E), 1)
    eidx = jnp.min(jnp.where(probs == pmax, lane, E), axis=1,
                   keepdims=True)
    one_hot = (lane == eidx).astype(jnp.float32)

    r = lax.broadcasted_iota(jnp.int32, (CHUNK, CHUNK), 0)
    c = lax.broadcasted_iota(jnp.int32, (CHUNK, CHUNK), 1)
    tril = (c <= r).astype(jnp.float32)
    carry = jnp.zeros((1, E), jnp.float32)
    for k in range(S // CHUNK):
        sl = slice(k * CHUNK, (k + 1) * CHUNK)
        oh = one_hot[sl]
        prio = lax.dot_general(tril, oh, (((1,), (0,)), ((), ())),
                               preferred_element_type=jnp.float32) + carry
        carry = carry + jnp.sum(oh, axis=0, keepdims=True)
        own = jnp.sum(prio * oh, axis=1, keepdims=True)
        own_i = own.astype(jnp.int32)
        valid = own_i <= CAP
        slot = eidx[sl] * CAP + own_i - 1
        tok = lax.broadcasted_iota(jnp.int32, (CHUNK, 1), 0) + k * CHUNK
        gidx_ref[sl] = jnp.where(valid, slot, NSLOT + tok)


def _router(x, ln_w, cls_w):
    return pl.pallas_call(
        _router_body,
        out_shape=(
            jax.ShapeDtypeStruct((S, D_MODEL), jnp.float32),
            jax.ShapeDtypeStruct((S, _LANES), jnp.float32),
            jax.ShapeDtypeStruct((S, 1), jnp.int32),
        ),
    )(x, ln_w, cls_w)


def _dispatch_body(fwd_hbm, gidx_hbm, xg_hbm, idx_v, rows_v, sem):
    wid = lax.axis_index("s") * 2 + lax.axis_index("c")
    base = wid * _TPW
    pltpu.sync_copy(gidx_hbm.at[pl.ds(base, _TPW)], idx_v)
    pltpu.sync_copy(fwd_hbm.at[pl.ds(base, _TPW)], rows_v)
    pltpu.async_copy(rows_v, xg_hbm.at[idx_v], sem).wait()


@functools.cache
def _dispatch():
    return pl.kernel(
        _dispatch_body,
        out_type=jax.ShapeDtypeStruct((TBL_ROWS, D_MODEL), jnp.float32),
        mesh=plsc.VectorSubcoreMesh(core_axis_name="c", subcore_axis_name="s"),
        scratch_types=[
            pltpu.VMEM((_TPW,), jnp.int32),
            pltpu.VMEM((_TPW, D_MODEL), jnp.float32),
            pltpu.SemaphoreType.DMA,
        ],
    )


def _experts_body(xg_ref, wia_ref, wib_ref, woa_ref, wob_ref, tbl_ref):
    x16 = xg_ref[...].astype(jnp.bfloat16)
    acc = None
    for wi_r, wo_r in ((wia_ref, woa_ref), (wib_ref, wob_ref)):
        h = lax.dot_general(x16, wi_r[0].astype(jnp.bfloat16),
                            (((1,), (1,)), ((), ())),
                            preferred_element_type=jnp.float32)
        h = jnp.maximum(h, 0.0).astype(jnp.bfloat16)
        part = lax.dot_general(h, wo_r[0].astype(jnp.bfloat16),
                               (((1,), (1,)), ((), ())),
                               preferred_element_type=jnp.float32)
        acc = part if acc is None else acc + part
    tbl_ref[...] = acc


def _experts(xg, wi, wo):
    return pl.pallas_call(
        _experts_body,
        grid=(E,),
        in_specs=[
            pl.BlockSpec((CAP, D_MODEL), lambda e: (e, 0)),
            pl.BlockSpec((1, FFB, D_MODEL), lambda e: (e, 0, 0)),
            pl.BlockSpec((1, FFB, D_MODEL), lambda e: (e, 1, 0)),
            pl.BlockSpec((1, D_MODEL, FFB), lambda e: (e, 0, 0)),
            pl.BlockSpec((1, D_MODEL, FFB), lambda e: (e, 0, 1)),
        ],
        out_specs=pl.BlockSpec((CAP, D_MODEL), lambda e: (e, 0)),
        out_shape=jax.ShapeDtypeStruct((TBL_ROWS, D_MODEL), jnp.float32),
        input_output_aliases={0: 0},
    )(xg, wi, wi, wo, wo)


_CCH = 16
_NCH = _TPW // _CCH


def _combine_body(tbl_hbm, hid_hbm, prep_hbm, gidx_hbm, out_hbm,
                  idx_v, rows_v, hid_v, prep_v, gsem, hsem, ssem):
    wid = lax.axis_index("s") * 2 + lax.axis_index("c")
    base = wid * _TPW
    pltpu.sync_copy(gidx_hbm.at[pl.ds(base, _TPW)], idx_v)
    pltpu.sync_copy(prep_hbm.at[pl.ds(base, _TPW)], prep_v)

    gets, hids, puts = {}, {}, {}

    def start(c):
        sl = pl.ds(c * _CCH, _CCH)
        gets[c] = pltpu.async_copy(tbl_hbm.at[idx_v.at[sl]],
                                   rows_v.at[sl], gsem)
        hids[c] = pltpu.async_copy(hid_hbm.at[pl.ds(base + c * _CCH, _CCH)],
                                   hid_v.at[sl], hsem)

    start(0)
    for c in range(_NCH):
        if c + 1 < _NCH:
            start(c + 1)
        gets[c].wait()
        hids[c].wait()

        def tok(i, _):
            p = prep_v[i, :]
            for j in range(D_MODEL // _LANES):
                sl = pl.ds(j * _LANES, _LANES)
                hid_v[i, sl] = hid_v[i, sl] + p * rows_v[i, sl]
            return _

        lax.fori_loop(c * _CCH, (c + 1) * _CCH, tok, None)
        puts[c] = pltpu.async_copy(hid_v.at[pl.ds(c * _CCH, _CCH)],
                                   out_hbm.at[pl.ds(base + c * _CCH, _CCH)],
                                   ssem)
    for c in range(_NCH):
        puts[c].wait()


@functools.cache
def _combine():
    return pl.kernel(
        _combine_body,
        out_type=jax.ShapeDtypeStruct((S, D_MODEL), jnp.float32),
        mesh=plsc.VectorSubcoreMesh(core_axis_name="c", subcore_axis_name="s"),
        scratch_types=[
            pltpu.VMEM((_TPW,), jnp.int32),
            pltpu.VMEM((_TPW, D_MODEL), jnp.float32),
            pltpu.VMEM((_TPW, D_MODEL), jnp.float32),
            pltpu.VMEM((_TPW, _LANES), jnp.float32),
            pltpu.SemaphoreType.DMA,
            pltpu.SemaphoreType.DMA,
            pltpu.SemaphoreType.DMA,
        ],
    )


def kernel(hidden_states, ln_weight, classifier_w, wi, wo):
    x = hidden_states.reshape(S, D_MODEL)
    fwd, prep, gidx = _router(x, ln_weight.reshape(1, D_MODEL), classifier_w)
    gidx = gidx.reshape(S)
    xg = _dispatch()(fwd, gidx)
    tbl = _experts(xg, wi, wo)
    out = _combine()(tbl, x, prep, gidx)
    return out.reshape(hidden_states.shape)


# speedup vs baseline: 1.0053x; 1.0053x over previous
import functools

import jax
import jax.numpy as jnp
from jax import lax
from jax.experimental import pallas as pl
from jax.experimental.pallas import tpu as pltpu
from jax.experimental.pallas import tpu_sc as plsc

S = 2048
D_MODEL = 768
D_FF = 2048
E = 8
CAP = 512
EPS = 1e-6
CHUNK = 256
FFB = 1024
NSLOT = E * CAP
TBL_ROWS = NSLOT + S

_LANES = 16
_NW = 32
_TPW = S // _NW


def _router_body(x_ref, lnw_ref, cls_ref, fwd_ref, prep_ref, gidx_ref):
    x = x_ref[...]
    var = jnp.mean(x * x, axis=1, keepdims=True)
    fwd = x * lax.rsqrt(var + EPS) * lnw_ref[...]
    fwd_ref[...] = fwd

    logits = lax.dot_general(fwd, cls_ref[...], (((1,), (1,)), ((), ())),
                             preferred_element_type=jnp.float32)
    m = jnp.max(logits, axis=1, keepdims=True)
    ex = jnp.exp(logits - m)
    probs = ex / jnp.sum(ex, axis=1, keepdims=True)
    pmax = jnp.max(probs, axis=1, keepdims=True)
    prep_ref[...] = jnp.broadcast_to(pmax, (S, _LANES))

    lane = lax.broadcasted_iota(jnp.int32, (S, E), 1)
    eidx = jnp.min(jnp.where(probs == pmax, lane, E), axis=1,
                   keepdims=True)
    one_hot = (lane == eidx).astype(jnp.float32)

    r = lax.broadcasted_iota(jnp.int32, (CHUNK, CHUNK), 0)
    c = lax.broadcasted_iota(jnp.int32, (CHUNK, CHUNK), 1)
    tril = (c <= r).astype(jnp.float32)
    carry = jnp.zeros((1, E), jnp.float32)
    for k in range(S // CHUNK):
        sl = slice(k * CHUNK, (k + 1) * CHUNK)
        oh = one_hot[sl]
        prio = lax.dot_general(tril, oh, (((1,), (0,)), ((), ())),
                               preferred_element_type=jnp.float32) + carry
        carry = carry + jnp.sum(oh, axis=0, keepdims=True)
        own = jnp.sum(prio * oh, axis=1, keepdims=True)
        own_i = own.astype(jnp.int32)
        valid = own_i <= CAP
        slot = eidx[sl] * CAP + own_i - 1
        tok = lax.broadcasted_iota(jnp.int32, (CHUNK, 1), 0) + k * CHUNK
        gidx_ref[sl] = jnp.where(valid, slot, NSLOT + tok)


def _router(x, ln_w, cls_w):
    return pl.pallas_call(
        _router_body,
        out_shape=(
            jax.ShapeDtypeStruct((S, D_MODEL), jnp.float32),
            jax.ShapeDtypeStruct((S, _LANES), jnp.float32),
            jax.ShapeDtypeStruct((S, 1), jnp.int32),
        ),
    )(x, ln_w, cls_w)


_DCH = 16
_DNCH = _TPW // _DCH


def _dispatch_body(fwd_hbm, gidx_hbm, xg_hbm, idx0, idx1, idx2, idx3,
                   rows_v, lsem, ssem):
    wid = lax.axis_index("s") * 2 + lax.axis_index("c")
    base = wid * _TPW
    idxs = (idx0, idx1, idx2, idx3)
    for c in range(_DNCH):
        pltpu.sync_copy(gidx_hbm.at[pl.ds(base + c * _DCH, _DCH)], idxs[c])

    loads, puts = {}, {}

    def start(c):
        loads[c] = pltpu.async_copy(
            fwd_hbm.at[pl.ds(base + c * _DCH, _DCH)],
            rows_v.at[pl.ds(c * _DCH, _DCH)], lsem)

    start(0)
    for c in range(_DNCH):
        if c + 1 < _DNCH:
            start(c + 1)
        loads[c].wait()
        puts[c] = pltpu.async_copy(rows_v.at[pl.ds(c * _DCH, _DCH)],
                                   xg_hbm.at[idxs[c]], ssem)
    for c in range(_DNCH):
        puts[c].wait()


@functools.cache
def _dispatch():
    return pl.kernel(
        _dispatch_body,
        out_type=jax.ShapeDtypeStruct((TBL_ROWS, D_MODEL), jnp.float32),
        mesh=plsc.VectorSubcoreMesh(core_axis_name="c", subcore_axis_name="s"),
        scratch_types=[
            pltpu.VMEM((_DCH,), jnp.int32),
            pltpu.VMEM((_DCH,), jnp.int32),
            pltpu.VMEM((_DCH,), jnp.int32),
            pltpu.VMEM((_DCH,), jnp.int32),
            pltpu.VMEM((_TPW, D_MODEL), jnp.float32),
            pltpu.SemaphoreType.DMA,
            pltpu.SemaphoreType.DMA,
        ],
    )


def _experts_body(xg_ref, wi_ref, wo_ref, tbl_ref):
    x16 = xg_ref[...].astype(jnp.bfloat16)
    h = lax.dot_general(x16, wi_ref[0].astype(jnp.bfloat16),
                        (((1,), (1,)), ((), ())),
                        preferred_element_type=jnp.float32)
    h = jnp.maximum(h, 0.0).astype(jnp.bfloat16)
    tbl_ref[...] = lax.dot_general(h, wo_ref[0].astype(jnp.bfloat16),
                                   (((1,), (1,)), ((), ())),
                                   preferred_element_type=jnp.float32)


def _experts(xg, wi, wo):
    return pl.pallas_call(
        _experts_body,
        grid=(E,),
        in_specs=[
            pl.BlockSpec((CAP, D_MODEL), lambda e: (e, 0)),
            pl.BlockSpec((1, D_FF, D_MODEL), lambda e: (e, 0, 0)),
            pl.BlockSpec((1, D_MODEL, D_FF), lambda e: (e, 0, 0)),
        ],
        out_specs=pl.BlockSpec((CAP, D_MODEL), lambda e: (e, 0)),
        out_shape=jax.ShapeDtypeStruct((TBL_ROWS, D_MODEL), jnp.float32),
        input_output_aliases={0: 0},
    )(xg, wi, wo)


_CCH = 16
_NCH = _TPW // _CCH


def _combine_body(tbl_hbm, hid_hbm, prep_hbm, gidx_hbm, out_hbm,
                  idx_v, rows_v, hid_v, prep_v, gsem, hsem, ssem):
    wid = lax.axis_index("s") * 2 + lax.axis_index("c")
    base = wid * _TPW
    pltpu.sync_copy(gidx_hbm.at[pl.ds(base, _TPW)], idx_v)
    pltpu.sync_copy(prep_hbm.at[pl.ds(base, _TPW)], prep_v)

    gets, hids, puts = {}, {}, {}

    def start(c):
        sl = pl.ds(c * _CCH, _CCH)
        gets[c] = pltpu.async_copy(tbl_hbm.at[idx_v.at[sl]],
                                   rows_v.at[sl], gsem)
        hids[c] = pltpu.async_copy(hid_hbm.at[pl.ds(base + c * _CCH, _CCH)],
                                   hid_v.at[sl], hsem)

    start(0)
    for c in range(_NCH):
        if c + 1 < _NCH:
            start(c + 1)
        gets[c].wait()
        hids[c].wait()

        def tok(i, _):
            p = prep_v[i, :]
            for j in range(D_MODEL // _LANES):
                sl = pl.ds(j * _LANES, _LANES)
                hid_v[i, sl] = hid_v[i, sl] + p * rows_v[i, sl]
            return _

        lax.fori_loop(c * _CCH, (c + 1) * _CCH, tok, None)
        puts[c] = pltpu.async_copy(hid_v.at[pl.ds(c * _CCH, _CCH)],
                                   out_hbm.at[pl.ds(base + c * _CCH, _CCH)],
                                   ssem)
    for c in range(_NCH):
        puts[c].wait()


@functools.cache
def _combine():
    return pl.kernel(
        _combine_body,
        out_type=jax.ShapeDtypeStruct((S, D_MODEL), jnp.float32),
        mesh=plsc.VectorSubcoreMesh(core_axis_name="c", subcore_axis_name="s"),
        scratch_types=[
            pltpu.VMEM((_TPW,), jnp.int32),
            pltpu.VMEM((_TPW, D_MODEL), jnp.float32),
            pltpu.VMEM((_TPW, D_MODEL), jnp.float32),
            pltpu.VMEM((_TPW, _LANES), jnp.float32),
            pltpu.SemaphoreType.DMA,
            pltpu.SemaphoreType.DMA,
            pltpu.SemaphoreType.DMA,
        ],
    )


def kernel(hidden_states, ln_weight, classifier_w, wi, wo):
    x = hidden_states.reshape(S, D_MODEL)
    fwd, prep, gidx = _router(x, ln_weight.reshape(1, D_MODEL), classifier_w)
    gidx = gidx.reshape(S)
    xg = _dispatch()(fwd, gidx)
    tbl = _experts(xg, wi, wo)
    out = _combine()(tbl, x, prep, gidx)
    return out.reshape(hidden_states.shape)


# speedup vs baseline: 1.0193x; 1.0140x over previous
import functools

import jax
import jax.numpy as jnp
from jax import lax
from jax.experimental import pallas as pl
from jax.experimental.pallas import tpu as pltpu
from jax.experimental.pallas import tpu_sc as plsc

S = 2048
D_MODEL = 768
D_FF = 2048
E = 8
CAP = 512
EPS = 1e-6
CHUNK = 256
FFB = 1024
NSLOT = E * CAP
TBL_ROWS = NSLOT + S

_LANES = 16
_NW = 32
_TPW = S // _NW


def _router_body(x_ref, lnw_ref, cls_ref, fwd_ref, prep_ref, gidx_ref):
    x = x_ref[...]
    var = jnp.mean(x * x, axis=1, keepdims=True)
    fwd = x * lax.rsqrt(var + EPS) * lnw_ref[...]
    fwd_ref[...] = fwd

    logits = lax.dot_general(fwd, cls_ref[...], (((1,), (1,)), ((), ())),
                             preferred_element_type=jnp.float32)
    m = jnp.max(logits, axis=1, keepdims=True)
    ex = jnp.exp(logits - m)
    probs = ex / jnp.sum(ex, axis=1, keepdims=True)
    pmax = jnp.max(probs, axis=1, keepdims=True)
    prep_ref[...] = jnp.broadcast_to(pmax, (S, _LANES))

    lane = lax.broadcasted_iota(jnp.int32, (S, E), 1)
    eidx = jnp.min(jnp.where(probs == pmax, lane, E), axis=1,
                   keepdims=True)
    one_hot = (lane == eidx).astype(jnp.float32)

    r = lax.broadcasted_iota(jnp.int32, (CHUNK, CHUNK), 0)
    c = lax.broadcasted_iota(jnp.int32, (CHUNK, CHUNK), 1)
    tril = (c <= r).astype(jnp.float32)
    carry = jnp.zeros((1, E), jnp.float32)
    for k in range(S // CHUNK):
        sl = slice(k * CHUNK, (k + 1) * CHUNK)
        oh = one_hot[sl]
        prio = lax.dot_general(tril, oh, (((1,), (0,)), ((), ())),
                               preferred_element_type=jnp.float32) + carry
        carry = carry + jnp.sum(oh, axis=0, keepdims=True)
        own = jnp.sum(prio * oh, axis=1, keepdims=True)
        own_i = own.astype(jnp.int32)
        valid = own_i <= CAP
        slot = eidx[sl] * CAP + own_i - 1
        tok = lax.broadcasted_iota(jnp.int32, (CHUNK, 1), 0) + k * CHUNK
        gidx_ref[sl] = jnp.where(valid, slot, NSLOT + tok)


def _router(x, ln_w, cls_w):
    return pl.pallas_call(
        _router_body,
        out_shape=(
            jax.ShapeDtypeStruct((S, D_MODEL), jnp.float32),
            jax.ShapeDtypeStruct((S, _LANES), jnp.float32),
            jax.ShapeDtypeStruct((S, 1), jnp.int32),
        ),
    )(x, ln_w, cls_w)


def _dispatch_body(fwd_hbm, gidx_hbm, xg_hbm, idx_v, rows_v, sem):
    wid = lax.axis_index("s") * 2 + lax.axis_index("c")
    base = wid * _TPW
    pltpu.sync_copy(gidx_hbm.at[pl.ds(base, _TPW)], idx_v)
    pltpu.sync_copy(fwd_hbm.at[pl.ds(base, _TPW)], rows_v)
    pltpu.async_copy(rows_v, xg_hbm.at[idx_v], sem).wait()


@functools.cache
def _dispatch():
    return pl.kernel(
        _dispatch_body,
        out_type=jax.ShapeDtypeStruct((TBL_ROWS, D_MODEL), jnp.float32),
        mesh=plsc.VectorSubcoreMesh(core_axis_name="c", subcore_axis_name="s"),
        scratch_types=[
            pltpu.VMEM((_TPW,), jnp.int32),
            pltpu.VMEM((_TPW, D_MODEL), jnp.float32),
            pltpu.SemaphoreType.DMA,
        ],
    )


def _experts_body(xg_ref, wi_ref, wo_ref, tbl_ref):
    x16 = xg_ref[...].astype(jnp.bfloat16)
    h = lax.dot_general(x16, wi_ref[0].astype(jnp.bfloat16),
                        (((1,), (1,)), ((), ())),
                        preferred_element_type=jnp.float32)
    h = jnp.maximum(h, 0.0).astype(jnp.bfloat16)
    tbl_ref[...] = lax.dot_general(h, wo_ref[0].astype(jnp.bfloat16),
                                   (((1,), (1,)), ((), ())),
                                   preferred_element_type=jnp.float32)


def _experts(xg, wi, wo):
    return pl.pallas_call(
        _experts_body,
        grid=(E,),
        in_specs=[
            pl.BlockSpec((CAP, D_MODEL), lambda e: (e, 0)),
            pl.BlockSpec((1, D_FF, D_MODEL), lambda e: (e, 0, 0)),
            pl.BlockSpec((1, D_MODEL, D_FF), lambda e: (e, 0, 0)),
        ],
        out_specs=pl.BlockSpec((CAP, D_MODEL), lambda e: (e, 0)),
        out_shape=jax.ShapeDtypeStruct((TBL_ROWS, D_MODEL), jnp.float32),
        input_output_aliases={0: 0},
    )(xg, wi, wo)


_CCH = 16
_NCH = _TPW // _CCH


def _combine_body(tbl_hbm, hid_hbm, prep_hbm, gidx_hbm, out_hbm,
                  idx_v, rows_v, hid_v, prep_v, gsem, hsem, ssem):
    wid = lax.axis_index("s") * 2 + lax.axis_index("c")
    base = wid * _TPW
    pltpu.sync_copy(gidx_hbm.at[pl.ds(base, _TPW)], idx_v)
    pltpu.sync_copy(prep_hbm.at[pl.ds(base, _TPW)], prep_v)

    gets, hids, puts = {}, {}, {}

    def start(c):
        sl = pl.ds(c * _CCH, _CCH)
        gets[c] = pltpu.async_copy(tbl_hbm.at[idx_v.at[sl]],
                                   rows_v.at[sl], gsem)
        hids[c] = pltpu.async_copy(hid_hbm.at[pl.ds(base + c * _CCH, _CCH)],
                                   hid_v.at[sl], hsem)

    start(0)
    for c in range(_NCH):
        if c + 1 < _NCH:
            start(c + 1)
        gets[c].wait()
        hids[c].wait()

        def tok(i, _):
            p = prep_v[i, :]
            for j in range(D_MODEL // _LANES):
                sl = pl.ds(j * _LANES, _LANES)
                hid_v[i, sl] = hid_v[i, sl] + p * rows_v[i, sl]
            return _

        lax.fori_loop(c * _CCH, (c + 1) * _CCH, tok, None)
        puts[c] = pltpu.async_copy(hid_v.at[pl.ds(c * _CCH, _CCH)],
                                   out_hbm.at[pl.ds(base + c * _CCH, _CCH)],
                                   ssem)
    for c in range(_NCH):
        puts[c].wait()


@functools.cache
def _combine():
    return pl.kernel(
        _combine_body,
        out_type=jax.ShapeDtypeStruct((S, D_MODEL), jnp.float32),
        mesh=plsc.VectorSubcoreMesh(core_axis_name="c", subcore_axis_name="s"),
        scratch_types=[
            pltpu.VMEM((_TPW,), jnp.int32),
            pltpu.VMEM((_TPW, D_MODEL), jnp.float32),
            pltpu.VMEM((_TPW, D_MODEL), jnp.float32),
            pltpu.VMEM((_TPW, _LANES), jnp.float32),
            pltpu.SemaphoreType.DMA,
            pltpu.SemaphoreType.DMA,
            pltpu.SemaphoreType.DMA,
        ],
    )


def kernel(hidden_states, ln_weight, classifier_w, wi, wo):
    x = hidden_states.reshape(S, D_MODEL)
    fwd, prep, gidx = _router(x, ln_weight.reshape(1, D_MODEL), classifier_w)
    gidx = gidx.reshape(S)
    xg = _dispatch()(fwd, gidx)
    tbl = _experts(xg, wi, wo)
    out = _combine()(tbl, x, prep, gidx)
    return out.reshape(hidden_states.shape)
